# baseline (speedup 1.0000x reference)
import functools

import jax
import jax.numpy as jnp
from jax import lax
from jax.experimental import pallas as pl
from jax.experimental.pallas import tpu as pltpu
from jax.experimental.pallas import tpu_sc as plsc

B, S, H = 4, 2048, 768
HIN = 128
N = B * S
NSPLIT = 4
NPART = N // NSPLIT
LN_EPS = 1e-12

NC, NS = 2, 16
NW = NC * NS
TOK_PER_W = NPART // NW
CHUNK = 16
NCHUNK = TOK_PER_W // CHUNK
LANES = 16


def _sc_gather3_sum(word_emb, tpos_emb, pos_emb, iw, itp, ip):
  mesh = plsc.VectorSubcoreMesh(core_axis_name="c", subcore_axis_name="s")

  @functools.partial(
      pl.kernel,
      mesh=mesh,
      out_type=jax.ShapeDtypeStruct((NPART, H), jnp.float32),
      scratch_types=[
          pltpu.VMEM((TOK_PER_W,), jnp.int32),
          pltpu.VMEM((TOK_PER_W,), jnp.int32),
          pltpu.VMEM((TOK_PER_W,), jnp.int32),
          [pltpu.VMEM((CHUNK, H), jnp.float32)] * 3,
          [pltpu.VMEM((CHUNK, H), jnp.float32)] * 3,
          pltpu.SemaphoreType.DMA,
          pltpu.SemaphoreType.DMA,
          pltpu.SemaphoreType.DMA,
          pltpu.SemaphoreType.DMA,
      ],
  )
  def k(word_hbm, tpe_hbm, pe_hbm, iw_hbm, itp_hbm, ip_hbm, out_hbm,
        iw_v, itp_v, ip_v, set0, set1, g0, g1, o0, o1):
    wid = lax.axis_index("s") * NC + lax.axis_index("c")
    base0 = wid * TOK_PER_W
    bufs = (set0, set1)
    gsem = (g0, g1)
    osem = (o0, o1)

    pltpu.sync_copy(iw_hbm.at[pl.ds(base0, TOK_PER_W)], iw_v)
    pltpu.sync_copy(itp_hbm.at[pl.ds(base0, TOK_PER_W)], itp_v)
    pltpu.sync_copy(ip_hbm.at[pl.ds(base0, TOK_PER_W)], ip_v)

    def fire(it, s):
      sl = pl.ds(it * CHUNK, CHUNK)
      return (
          pltpu.async_copy(word_hbm.at[iw_v.at[sl]], bufs[s][0], gsem[s]),
          pltpu.async_copy(tpe_hbm.at[itp_v.at[sl]], bufs[s][1], gsem[s]),
          pltpu.async_copy(pe_hbm.at[ip_v.at[sl]], bufs[s][2], gsem[s]),
      )

    gdesc = [None, None]
    odesc = [None, None]
    gdesc[0] = fire(0, 0)
    for it in range(NCHUNK):
      s = it % 2
      ss = 1 - s
      for dsc in gdesc[s]:
        dsc.wait()
      if it + 1 < NCHUNK:
        if odesc[ss] is not None:
          odesc[ss].wait()
        gdesc[ss] = fire(it + 1, ss)
      b0, b1, b2 = bufs[s]

      def row(j, c2, b0=b0, b1=b1, b2=b2):
        for kk in range(H // LANES):
          sl2 = pl.ds(kk * LANES, LANES)
          plsc.addupdate(b0.at[j, sl2], b1[j, sl2] + b2[j, sl2])
        return c2

      lax.fori_loop(0, CHUNK, row, 0, unroll=False)
      odesc[s] = pltpu.async_copy(
          b0, out_hbm.at[pl.ds(base0 + it * CHUNK, CHUNK)], osem[s])
    odesc[0].wait()
    odesc[1].wait()

  return k(word_emb, tpos_emb, pos_emb, iw, itp, ip)


BT = 512
GRID = NPART // BT


def _tc_fuse_body(vm_ref, mh_ref, part_ref, ttm_ref, vW_ref, mW_ref,
                  bias_ref, ttd_ref, gam_ref, bet_ref, *rest):
  out_ref = rest[-1]
  x = jnp.dot(vm_ref[...], vW_ref[...], preferred_element_type=jnp.float32)
  x = x + jnp.dot(mh_ref[...], mW_ref[...], preferred_element_type=jnp.float32)
  x = x + part_ref[...]
  x = x + bias_ref[...]
  x = x + ttm_ref[...] * ttd_ref[...]
  mu = jnp.mean(x, axis=-1, keepdims=True)
  xc = x - mu
  var = jnp.mean(xc * xc, axis=-1, keepdims=True)
  y = xc * lax.rsqrt(var + LN_EPS)
  out_ref[...] = y * gam_ref[...] + bet_ref[...]


def _tc_fuse_half(off, vm, mh, partial, ttm, vW, mW, bias, ttd, gam, bet,
                  prev):
  in_specs = [
      pl.BlockSpec((BT, H), lambda i: (i + off, 0)),
      pl.BlockSpec((BT, HIN), lambda i: (i + off, 0)),
      pl.BlockSpec((BT, H), lambda i: (i, 0)),
      pl.BlockSpec((BT, 1), lambda i: (i + off, 0)),
      pl.BlockSpec((H, H), lambda i: (0, 0)),
      pl.BlockSpec((HIN, H), lambda i: (0, 0)),
      pl.BlockSpec((1, H), lambda i: (0, 0)),
      pl.BlockSpec((1, H), lambda i: (0, 0)),
      pl.BlockSpec((1, H), lambda i: (0, 0)),
      pl.BlockSpec((1, H), lambda i: (0, 0)),
  ]
  args = [vm, mh, partial, ttm, vW, mW, bias, ttd, gam, bet]
  aliases = {}
  if prev is not None:
    in_specs.append(pl.BlockSpec(memory_space=pltpu.MemorySpace.HBM))
    args.append(prev)
    aliases = {10: 0}
  return pl.pallas_call(
      _tc_fuse_body,
      grid=(GRID,),
      in_specs=in_specs,
      out_specs=pl.BlockSpec((BT, H), lambda i: (i + off, 0)),
      out_shape=jax.ShapeDtypeStruct((N, H), jnp.float32),
      input_output_aliases=aliases,
      compiler_params=pltpu.CompilerParams(
          dimension_semantics=("arbitrary",),
      ),
  )(*args)


def kernel(input_ids, token_type_ids, position_ids, token_position_ids,
           value_ids, minhash_vals, word_emb, token_type_emb,
           token_position_emb, position_emb, value_W, value_b, minhash_W,
           minhash_b, ln_gamma, ln_beta):
  iw = input_ids.reshape(N).astype(jnp.int32)
  itp = token_position_ids.reshape(N).astype(jnp.int32)
  ip = position_ids.reshape(N).astype(jnp.int32)

  parts = [
      _sc_gather3_sum(word_emb, token_position_emb, position_emb,
                      iw[h * NPART:(h + 1) * NPART],
                      itp[h * NPART:(h + 1) * NPART],
                      ip[h * NPART:(h + 1) * NPART])
      for h in range(NSPLIT)
  ]

  ttm = token_type_ids.reshape(N, 1).astype(jnp.float32)
  bias = (value_b + minhash_b + token_type_emb[0]).reshape(1, H)
  ttd = (token_type_emb[1] - token_type_emb[0]).reshape(1, H)
  vm = value_ids.reshape(N, H)
  mh = minhash_vals.reshape(N, HIN)
  gam = ln_gamma.reshape(1, H)
  bet = ln_beta.reshape(1, H)

  out = None
  for h in range(NSPLIT):
    out = _tc_fuse_half(h * GRID, vm, mh, parts[h], ttm, value_W, minhash_W,
                        bias, ttd, gam, bet, out)
  return out.reshape(B, S, H)


# speedup vs baseline: 1.0626x; 1.0626x over previous
import functools

import jax
import jax.numpy as jnp
from jax import lax
from jax.experimental import pallas as pl
from jax.experimental.pallas import tpu as pltpu
from jax.experimental.pallas import tpu_sc as plsc

B, S, H = 4, 2048, 768
HIN = 128
N = B * S
STAGES = ((0, 5120), (5120, 2048), (7168, 1024))
LN_EPS = 1e-12

NC, NS = 2, 16
NW = NC * NS
CHUNK = 16
LANES = 16


def _sc_gather3_sum(word_emb, tpos_emb, pos_emb, iw, itp, ip, tok_off,
                    ntok):
  tok_per_w = ntok // NW
  nchunk = tok_per_w // CHUNK
  mesh = plsc.VectorSubcoreMesh(core_axis_name="c", subcore_axis_name="s")

  @functools.partial(
      pl.kernel,
      mesh=mesh,
      out_type=jax.ShapeDtypeStruct((ntok, H), jnp.float32),
      scratch_types=[
          pltpu.VMEM((tok_per_w,), jnp.int32),
          pltpu.VMEM((tok_per_w,), jnp.int32),
          pltpu.VMEM((tok_per_w,), jnp.int32),
          [pltpu.VMEM((CHUNK, H), jnp.float32)] * 3,
          [pltpu.VMEM((CHUNK, H), jnp.float32)] * 3,
          pltpu.SemaphoreType.DMA,
          pltpu.SemaphoreType.DMA,
          pltpu.SemaphoreType.DMA,
          pltpu.SemaphoreType.DMA,
      ],
  )
  def k(word_hbm, tpe_hbm, pe_hbm, iw_hbm, itp_hbm, ip_hbm, out_hbm,
        iw_v, itp_v, ip_v, set0, set1, g0, g1, o0, o1):
    wid = lax.axis_index("s") * NC + lax.axis_index("c")
    lbase = wid * tok_per_w
    hbase = tok_off + lbase
    bufs = (set0, set1)
    gsem = (g0, g1)
    osem = (o0, o1)

    pltpu.sync_copy(iw_hbm.at[pl.ds(hbase, tok_per_w)], iw_v)
    pltpu.sync_copy(itp_hbm.at[pl.ds(hbase, tok_per_w)], itp_v)
    pltpu.sync_copy(ip_hbm.at[pl.ds(hbase, tok_per_w)], ip_v)

    def fire(it, s):
      sl = pl.ds(it * CHUNK, CHUNK)
      return (
          pltpu.async_copy(word_hbm.at[iw_v.at[sl]], bufs[s][0], gsem[s]),
          pltpu.async_copy(tpe_hbm.at[itp_v.at[sl]], bufs[s][1], gsem[s]),
          pltpu.async_copy(pe_hbm.at[ip_v.at[sl]], bufs[s][2], gsem[s]),
      )

    gdesc = [None, None]
    odesc = [None, None]
    gdesc[0] = fire(0, 0)
    for it in range(nchunk):
      s = it % 2
      ss = 1 - s
      for dsc in gdesc[s]:
        dsc.wait()
      if it + 1 < nchunk:
        if odesc[ss] is not None:
          odesc[ss].wait()
        gdesc[ss] = fire(it + 1, ss)
      b0, b1, b2 = bufs[s]

      def row(j, c2, b0=b0, b1=b1, b2=b2):
        for kk in range(H // LANES):
          sl2 = pl.ds(kk * LANES, LANES)
          plsc.addupdate(b0.at[j, sl2], b1[j, sl2] + b2[j, sl2])
        return c2

      lax.fori_loop(0, CHUNK, row, 0, unroll=False)
      odesc[s] = pltpu.async_copy(
          b0, out_hbm.at[pl.ds(lbase + it * CHUNK, CHUNK)], osem[s])
    odesc[0].wait()
    if odesc[1] is not None:
      odesc[1].wait()

  return k(word_emb, tpos_emb, pos_emb, iw, itp, ip)


BT = 512


def _tc_fuse_body(vm_ref, mh_ref, part_ref, ttm_ref, vW_ref, mW_ref,
                  bias_ref, ttd_ref, gam_ref, bet_ref, *rest):
  out_ref = rest[-1]
  x = jnp.dot(vm_ref[...], vW_ref[...], preferred_element_type=jnp.float32)
  x = x + jnp.dot(mh_ref[...], mW_ref[...], preferred_element_type=jnp.float32)
  x = x + part_ref[...]
  x = x + bias_ref[...]
  x = x + ttm_ref[...] * ttd_ref[...]
  mu = jnp.mean(x, axis=-1, keepdims=True)
  xc = x - mu
  var = jnp.mean(xc * xc, axis=-1, keepdims=True)
  y = xc * lax.rsqrt(var + LN_EPS)
  out_ref[...] = y * gam_ref[...] + bet_ref[...]


def _tc_fuse_half(off, grid, vm, mh, partial, ttm, vW, mW, bias, ttd, gam,
                  bet, prev):
  in_specs = [
      pl.BlockSpec((BT, H), lambda i: (i + off, 0)),
      pl.BlockSpec((BT, HIN), lambda i: (i + off, 0)),
      pl.BlockSpec((BT, H), lambda i: (i, 0)),
      pl.BlockSpec((BT, 1), lambda i: (i + off, 0)),
      pl.BlockSpec((H, H), lambda i: (0, 0)),
      pl.BlockSpec((HIN, H), lambda i: (0, 0)),
      pl.BlockSpec((1, H), lambda i: (0, 0)),
      pl.BlockSpec((1, H), lambda i: (0, 0)),
      pl.BlockSpec((1, H), lambda i: (0, 0)),
      pl.BlockSpec((1, H), lambda i: (0, 0)),
  ]
  args = [vm, mh, partial, ttm, vW, mW, bias, ttd, gam, bet]
  aliases = {}
  if prev is not None:
    in_specs.append(pl.BlockSpec(memory_space=pltpu.MemorySpace.HBM))
    args.append(prev)
    aliases = {10: 0}
  return pl.pallas_call(
      _tc_fuse_body,
      grid=(grid,),
      in_specs=in_specs,
      out_specs=pl.BlockSpec((BT, H), lambda i: (i + off, 0)),
      out_shape=jax.ShapeDtypeStruct((N, H), jnp.float32),
      input_output_aliases=aliases,
      compiler_params=pltpu.CompilerParams(
          dimension_semantics=("arbitrary",),
      ),
  )(*args)


def kernel(input_ids, token_type_ids, position_ids, token_position_ids,
           value_ids, minhash_vals, word_emb, token_type_emb,
           token_position_emb, position_emb, value_W, value_b, minhash_W,
           minhash_b, ln_gamma, ln_beta):
  iw = input_ids.reshape(N).astype(jnp.int32)
  itp = token_position_ids.reshape(N).astype(jnp.int32)
  ip = position_ids.reshape(N).astype(jnp.int32)

  parts = [
      _sc_gather3_sum(word_emb, token_position_emb, position_emb,
                      iw, itp, ip, off, ntok)
      for off, ntok in STAGES
  ]

  ttm = token_type_ids.reshape(N, 1).astype(jnp.float32)
  bias = (value_b + minhash_b + token_type_emb[0]).reshape(1, H)
  ttd = (token_type_emb[1] - token_type_emb[0]).reshape(1, H)
  vm = value_ids.reshape(N, H)
  mh = minhash_vals.reshape(N, HIN)
  gam = ln_gamma.reshape(1, H)
  bet = ln_beta.reshape(1, H)

  out = None
  for (off, ntok), part in zip(STAGES, parts):
    out = _tc_fuse_half(off // BT, ntok // BT, vm, mh, part, ttm, value_W,
                        minhash_W, bias, ttd, gam, bet, out)
  return out.reshape(B, S, H)


# speedup vs baseline: 1.0741x; 1.0108x over previous
import functools

import jax
import jax.numpy as jnp
from jax import lax
from jax.experimental import pallas as pl
from jax.experimental.pallas import tpu as pltpu
from jax.experimental.pallas import tpu_sc as plsc

B, S, H = 4, 2048, 768
HIN = 128
N = B * S
STAGES = ((0, 5120), (5120, 2048), (7168, 1024))
LN_EPS = 1e-12

NC, NS = 2, 16
NW = NC * NS
CHUNK = 16
LANES = 16


def _sc_gather3_sum(word_emb, tpos_emb, pos_emb, iw, itp, ip, tok_off,
                    ntok):
  tok_per_w = ntok // NW
  nchunk = tok_per_w // CHUNK
  mesh = plsc.VectorSubcoreMesh(core_axis_name="c", subcore_axis_name="s")

  @functools.partial(
      pl.kernel,
      mesh=mesh,
      out_type=jax.ShapeDtypeStruct((ntok, H), jnp.float32),
      scratch_types=[
          pltpu.VMEM((tok_per_w,), jnp.int32),
          pltpu.VMEM((tok_per_w,), jnp.int32),
          pltpu.VMEM((tok_per_w,), jnp.int32),
          [pltpu.VMEM((CHUNK, H), jnp.float32)] * 3,
          [pltpu.VMEM((CHUNK, H), jnp.float32)] * 3,
          pltpu.SemaphoreType.DMA,
          pltpu.SemaphoreType.DMA,
          pltpu.SemaphoreType.DMA,
          pltpu.SemaphoreType.DMA,
      ],
  )
  def k(word_hbm, tpe_hbm, pe_hbm, iw_hbm, itp_hbm, ip_hbm, out_hbm,
        iw_v, itp_v, ip_v, set0, set1, g0, g1, o0, o1):
    wid = lax.axis_index("s") * NC + lax.axis_index("c")
    lbase = wid * tok_per_w
    hbase = tok_off + lbase
    bufs = (set0, set1)
    gsem = (g0, g1)
    osem = (o0, o1)

    pltpu.sync_copy(iw_hbm.at[pl.ds(hbase, tok_per_w)], iw_v)
    pltpu.sync_copy(itp_hbm.at[pl.ds(hbase, tok_per_w)], itp_v)
    pltpu.sync_copy(ip_hbm.at[pl.ds(hbase, tok_per_w)], ip_v)

    def fire(it, s):
      sl = pl.ds(it * CHUNK, CHUNK)
      return (
          pltpu.async_copy(word_hbm.at[iw_v.at[sl]], bufs[s][0], gsem[s]),
          pltpu.async_copy(tpe_hbm.at[itp_v.at[sl]], bufs[s][1], gsem[s]),
          pltpu.async_copy(pe_hbm.at[ip_v.at[sl]], bufs[s][2], gsem[s]),
      )

    gdesc = [None, None]
    odesc = [None, None]
    gdesc[0] = fire(0, 0)
    for it in range(nchunk):
      s = it % 2
      ss = 1 - s
      for dsc in gdesc[s]:
        dsc.wait()
      if it + 1 < nchunk:
        if odesc[ss] is not None:
          odesc[ss].wait()
        gdesc[ss] = fire(it + 1, ss)
      b0, b1, b2 = bufs[s]

      def rowfn(j, c2, b0=b0, b1=b1, b2=b2):
        for kk in range(H // LANES):
          sl2 = pl.ds(kk * LANES, LANES)
          b0[j, sl2] = b0[j, sl2] + b1[j, sl2] + b2[j, sl2]
        return c2

      lax.fori_loop(0, CHUNK, rowfn, 0, unroll=False)
      obase = lbase + it * CHUNK
      odesc[s] = pltpu.async_copy(
          b0, out_hbm.at[pl.ds(obase, CHUNK)], osem[s])
    odesc[0].wait()
    if odesc[1] is not None:
      odesc[1].wait()

  return k(word_emb, tpos_emb, pos_emb, iw, itp, ip)


BT = 512


def _tc_fuse_body(vm_ref, mh_ref, part_ref, ttm_ref, vW_ref, mW_ref,
                  bias_ref, ttd_ref, gam_ref, bet_ref, *rest):
  out_ref = rest[-1]
  x = jnp.dot(vm_ref[...], vW_ref[...], preferred_element_type=jnp.float32)
  x = x + jnp.dot(mh_ref[...], mW_ref[...], preferred_element_type=jnp.float32)
  x = x + part_ref[...]
  x = x + bias_ref[...]
  x = x + ttm_ref[...] * ttd_ref[...]
  mu = jnp.mean(x, axis=-1, keepdims=True)
  xc = x - mu
  var = jnp.mean(xc * xc, axis=-1, keepdims=True)
  y = xc * lax.rsqrt(var + LN_EPS)
  out_ref[...] = y * gam_ref[...] + bet_ref[...]


def _tc_fuse_half(off, grid, vm, mh, partial, ttm, vW, mW, bias, ttd, gam,
                  bet, prev):
  in_specs = [
      pl.BlockSpec((BT, H), lambda i: (i + off, 0)),
      pl.BlockSpec((BT, HIN), lambda i: (i + off, 0)),
      pl.BlockSpec((BT, H), lambda i: (i, 0)),
      pl.BlockSpec((BT, 1), lambda i: (i + off, 0)),
      pl.BlockSpec((H, H), lambda i: (0, 0)),
      pl.BlockSpec((HIN, H), lambda i: (0, 0)),
      pl.BlockSpec((1, H), lambda i: (0, 0)),
      pl.BlockSpec((1, H), lambda i: (0, 0)),
      pl.BlockSpec((1, H), lambda i: (0, 0)),
      pl.BlockSpec((1, H), lambda i: (0, 0)),
  ]
  args = [vm, mh, partial, ttm, vW, mW, bias, ttd, gam, bet]
  aliases = {}
  if prev is not None:
    in_specs.append(pl.BlockSpec(memory_space=pltpu.MemorySpace.HBM))
    args.append(prev)
    aliases = {10: 0}
  return pl.pallas_call(
      _tc_fuse_body,
      grid=(grid,),
      in_specs=in_specs,
      out_specs=pl.BlockSpec((BT, H), lambda i: (i + off, 0)),
      out_shape=jax.ShapeDtypeStruct((N, H), jnp.float32),
      input_output_aliases=aliases,
      compiler_params=pltpu.CompilerParams(
          dimension_semantics=("arbitrary",),
      ),
  )(*args)


def kernel(input_ids, token_type_ids, position_ids, token_position_ids,
           value_ids, minhash_vals, word_emb, token_type_emb,
           token_position_emb, position_emb, value_W, value_b, minhash_W,
           minhash_b, ln_gamma, ln_beta):
  iw = input_ids.reshape(N).astype(jnp.int32)
  itp = token_position_ids.reshape(N).astype(jnp.int32)
  ip = position_ids.reshape(N).astype(jnp.int32)

  parts = [
      _sc_gather3_sum(word_emb, token_position_emb, position_emb,
                      iw, itp, ip, off, ntok)
      for off, ntok in STAGES
  ]

  ttm = token_type_ids.reshape(N, 1).astype(jnp.float32)
  bias = (value_b + minhash_b + token_type_emb[0]).reshape(1, H)
  ttd = (token_type_emb[1] - token_type_emb[0]).reshape(1, H)
  vm = value_ids.reshape(N, H)
  mh = minhash_vals.reshape(N, HIN)
  gam = ln_gamma.reshape(1, H)
  bet = ln_beta.reshape(1, H)

  out = None
  for (off, ntok), part in zip(STAGES, parts):
    out = _tc_fuse_half(off // BT, ntok // BT, vm, mh, part, ttm, value_W,
                        minhash_W, bias, ttd, gam, bet, out)
  return out.reshape(B, S, H)
